# baseline (speedup 1.0000x reference)
import functools

import jax
import jax.numpy as jnp
from jax import lax
from jax.experimental import pallas as pl
from jax.experimental.pallas import tpu as pltpu
from jax.experimental.pallas import tpu_sc as plsc

NEG_SLOPE = 0.2
NC = 2
NS = 16
LANES = 16
CHUNK = 64
DEPTH = 2
IDEPTH = 4
EDGE_UNROLL = 4
SCW = 144


def _proj_body(x_ref, wp_ref, bp_ref, wl_ref, wr_ref, h_ref, hl_ref, hr_ref):
    h = jnp.dot(x_ref[...], wp_ref[...],
                preferred_element_type=jnp.float32) + bp_ref[...]
    h_ref[...] = h
    hl_ref[...] = jnp.dot(h, wl_ref[...],
                          preferred_element_type=jnp.float32).astype(
                              jnp.bfloat16)
    hr_ref[...] = jnp.dot(h, wr_ref[...],
                          preferred_element_type=jnp.float32).astype(
                              jnp.bfloat16)


def _proj(x, wp, bp2, wl, wr):
    n, d = x.shape
    blk = 1000
    grid = n // blk
    row_spec = pl.BlockSpec((blk, d), lambda i: (i, 0))
    w_spec = pl.BlockSpec((d, d), lambda i: (0, 0))
    b_spec = pl.BlockSpec((1, d), lambda i: (0, 0))
    return pl.pallas_call(
        _proj_body,
        grid=(grid,),
        in_specs=[row_spec, w_spec, b_spec, w_spec, w_spec],
        out_specs=[row_spec, row_spec, row_spec],
        out_shape=[jax.ShapeDtypeStruct((n, d), jnp.float32),
                   jax.ShapeDtypeStruct((n, d), jnp.bfloat16),
                   jax.ShapeDtypeStruct((n, d), jnp.bfloat16)],
    )(x, wp, bp2, wl, wr)


def _sc_edge_body(hl_hbm, hr_hbm, idx_hbm, attb_hbm,
                  acc_out,
                  idx_c, hlr, hrr, scr, attb_v,
                  acc_sh, sin, sout, sidx):
    rows_per_tile = acc_sh.shape[0] // NS
    n_chunks = idx_hbm.shape[1]
    d = hl_hbm.shape[1]

    c = lax.axis_index("c")
    s = lax.axis_index("s")
    wid = c * NS + s

    pltpu.sync_copy(attb_hbm, attb_v)

    zeros = jnp.zeros((LANES,), jnp.float32)

    for b in range(DEPTH):
        def _z_rows(i, _):
            for g in range(SCW // LANES):
                scr[b][i, pl.ds(g * LANES, LANES)] = zeros
            return 0
        lax.fori_loop(0, CHUNK, _z_rows, 0)

    for j in range(rows_per_tile // CHUNK):
        base = s * rows_per_tile + j * CHUNK
        pltpu.sync_copy(scr[0], acc_sh.at[pl.ds(base, CHUNK)])
    plsc.subcore_barrier()

    av = [attb_v[g] for g in range(d // LANES)]

    def _issue_idx(cc, q):
        pltpu.async_copy(idx_hbm.at[wid, cc], idx_c[q], sidx[q])

    def _drain_idx(q):
        pltpu.make_async_copy(idx_hbm.at[wid, 0], idx_c[q], sidx[q]).wait()

    def _issue_gather(q, b):
        pltpu.async_copy(hl_hbm.at[idx_c[q].at[0]], hlr[b], sin[b])
        pltpu.async_copy(hr_hbm.at[idx_c[q].at[1]], hrr[b], sin[b])

    def _drain_gather(b):
        pltpu.make_async_copy(hl_hbm.at[pl.ds(0, CHUNK)], hlr[b],
                              sin[b]).wait()
        pltpu.make_async_copy(hl_hbm.at[pl.ds(0, CHUNK)], hrr[b],
                              sin[b]).wait()

    def _issue_scatter(q, b):
        pltpu.async_copy(scr[b], acc_sh.at[idx_c[q].at[1]], sout[b],
                         add=True)

    def _drain_scatter(b):
        pltpu.make_async_copy(acc_out.at[0, pl.ds(0, CHUNK)], scr[b],
                              sout[b]).wait()

    iota16 = lax.iota(jnp.int32, LANES)
    colidx = [32 * g + 2 * iota16 + p for g in range(d // 32)
              for p in range(2)]
    ecol = jnp.full((LANES,), d, jnp.int32)

    def _compute(b):
        def _edge(e4, _):
            for k in range(EDGE_UNROLL):
                e = e4 * EDGE_UNROLL + k
                acc = zeros
                hkeep = []
                for g in range(d // 32):
                    hv = hlr[b][e, pl.ds(g * 2 * LANES, 2 * LANES)]
                    rv = hrr[b][e, pl.ds(g * 2 * LANES, 2 * LANES)]
                    ha, hb = plsc.unpack(
                        hv, format=plsc.PackFormat.INTERLEAVED,
                        preferred_element_type=jnp.float32)
                    ra, rb = plsc.unpack(
                        rv, format=plsc.PackFormat.INTERLEAVED,
                        preferred_element_type=jnp.float32)
                    hkeep += [ha, hb]
                    za = ha + ra
                    zb = hb + rb
                    la = jnp.maximum(za, NEG_SLOPE * za)
                    lb = jnp.maximum(zb, NEG_SLOPE * zb)
                    acc = acc + la * av[2 * g] + lb * av[2 * g + 1]
                ee = jnp.exp(jnp.full((LANES,), jnp.sum(acc), jnp.float32))
                erow = jnp.full((LANES,), e, jnp.int32)
                for g in range(d // LANES):
                    plsc.store_scatter(scr[b], [erow, colidx[g]],
                                       hkeep[g] * ee)
                plsc.store_scatter(scr[b], [erow, ecol], ee)
            return 0

        lax.fori_loop(0, CHUNK // EDGE_UNROLL, _edge, 0)

    _issue_idx(0, 0)
    _issue_idx(1, 1)
    _drain_idx(0)
    _issue_gather(0, 0)

    def _round(i, _):
        for b4 in range(IDEPTH):
            cc = i * IDEPTH + b4
            rb = b4 % DEPTH
            nrb = (rb + 1) % DEPTH

            @pl.when(cc >= 2)
            def _():
                _drain_scatter(rb)

            @pl.when(cc + 1 < n_chunks)
            def _():
                _drain_idx((b4 + 1) % IDEPTH)
                _issue_gather((b4 + 1) % IDEPTH, nrb)

            @pl.when(cc + 2 < n_chunks)
            def _():
                _issue_idx(cc + 2, (b4 + 2) % IDEPTH)

            _drain_gather(rb)
            _compute(rb)
            _issue_scatter(b4, rb)
        return 0

    lax.fori_loop(0, n_chunks // IDEPTH, _round, 0)
    _drain_scatter((n_chunks - 2) % DEPTH)
    _drain_scatter((n_chunks - 1) % DEPTH)
    plsc.subcore_barrier()

    for j in range(rows_per_tile // CHUNK):
        base = s * rows_per_tile + j * CHUNK
        pltpu.sync_copy(acc_sh.at[pl.ds(base, CHUNK)], scr[0])
        pltpu.sync_copy(scr[0], acc_out.at[c, pl.ds(base, CHUNK)])


def _sc_edges(hl, hr, idx4, attb, npad):
    n, d = hl.shape
    rows_per_tile = npad // NS
    mesh = plsc.VectorSubcoreMesh(core_axis_name="c", subcore_axis_name="s",
                                  num_cores=NC, num_subcores=NS)
    f = pl.kernel(
        _sc_edge_body,
        out_type=jax.ShapeDtypeStruct((NC, npad, SCW), jnp.float32),
        mesh=mesh,
        compiler_params=pltpu.CompilerParams(
            needs_layout_passes=False, use_tc_tiling_on_sc=False),
        scratch_types=[
            [pltpu.VMEM((2, CHUNK), jnp.int32)] * IDEPTH,
            [pltpu.VMEM((CHUNK, d), jnp.bfloat16)] * DEPTH,
            [pltpu.VMEM((CHUNK, d), jnp.bfloat16)] * DEPTH,
            [pltpu.VMEM((CHUNK, SCW), jnp.float32)] * DEPTH,
            pltpu.VMEM((d // LANES, LANES), jnp.float32),
            pltpu.VMEM_SHARED((npad, SCW), jnp.float32),
            [pltpu.SemaphoreType.DMA] * DEPTH,
            [pltpu.SemaphoreType.DMA] * DEPTH,
            [pltpu.SemaphoreType.DMA] * IDEPTH,
        ],
    )
    return f(hl, hr, idx4, attb)


def _out_body(a0_ref, a1_ref, h_ref, bias_ref, wh_ref, bh_ref, o_ref):
    a = a0_ref[0] + a1_ref[0]
    d = h_ref.shape[1]
    acc = a[:, :d]
    den = a[:, d:d + 1]
    o = acc / (den + 1e-16) + h_ref[...] + bias_ref[...]
    o_ref[...] = jnp.dot(o, wh_ref[...],
                         preferred_element_type=jnp.float32) + bh_ref[...]


def _combine(accs, h, bias2, whp, bhp2):
    n, d = h.shape
    blk = 1000
    grid = n // blk
    row_spec = pl.BlockSpec((blk, d), lambda i: (i, 0))
    a0_spec = pl.BlockSpec((1, blk, SCW), lambda i: (0, i, 0))
    a1_spec = pl.BlockSpec((1, blk, SCW), lambda i: (1, i, 0))
    w_spec = pl.BlockSpec((d, d), lambda i: (0, 0))
    b_spec = pl.BlockSpec((1, d), lambda i: (0, 0))
    return pl.pallas_call(
        _out_body,
        grid=(grid,),
        in_specs=[a0_spec, a1_spec, row_spec, b_spec, w_spec, b_spec],
        out_specs=row_spec,
        out_shape=jax.ShapeDtypeStruct((n, d), jnp.float32),
    )(accs, accs, h, bias2, whp, bhp2)


def kernel(x, adj, Wp, bp, Wl, Wr, att, bias, Wh, bh):
    n, d = x.shape
    e = adj.shape[1]
    nw = NC * NS
    npad = ((n + NS * CHUNK - 1) // (NS * CHUNK)) * (NS * CHUNK)
    quantum = nw * CHUNK * IDEPTH
    epad = ((e + quantum - 1) // quantum) * quantum
    src_flat = jnp.concatenate(
        [adj[0], jnp.zeros((epad - e,), jnp.int32)])
    dst_flat = jnp.concatenate(
        [adj[1], jnp.full((epad - e,), npad - 1, jnp.int32)])
    n_chunks = epad // (nw * CHUNK)
    idx4 = jnp.stack([src_flat.reshape(nw, n_chunks, CHUNK),
                      dst_flat.reshape(nw, n_chunks, CHUNK)], axis=2)
    attb = att.reshape(d // 32, LANES, 2).transpose(0, 2, 1).reshape(
        d // LANES, LANES)

    h, hl, hr = _proj(x, Wp, bp.reshape(1, d), Wl, Wr)
    accs = _sc_edges(hl, hr, idx4, attb, npad)

    whp = jnp.pad(Wh, ((0, 0), (0, d - Wh.shape[1])))
    bhp2 = jnp.pad(bh, (0, d - bh.shape[0])).reshape(1, d)
    out = _combine(accs, h, bias.reshape(1, d), whp, bhp2)
    return out[:, :1]
